# Initial kernel scaffold; baseline (speedup 1.0000x reference)
import jax
import jax.numpy as jnp
from jax.experimental import pallas as pl


def kernel(h_in, W1, b1, W2, b2, W3, b3, Wp, bp, Wpi, bpi):
    raise NotImplementedError("write your pallas kernel here")


# speedup vs baseline: 2.6171x; 2.6171x over previous
import functools

import jax
import jax.numpy as jnp
from jax.experimental import pallas as pl
from jax.experimental.pallas import tpu as pltpu

L2 = 13
K = 2 ** L2
VQ = 64
HID = 1024
EMB = 768
B = 16
T = 576

KBLK = 1024
CHUNK = 1024

_dot = functools.partial(jnp.dot, preferred_element_type=jnp.float32,
                         precision=jax.lax.Precision.DEFAULT)


def _embed_kernel(w1t_ref, b1_ref, w2t_ref, b2_ref, w3t_ref, b3_ref,
                  et_ref):
    i = pl.program_id(0)
    row0 = i * KBLK
    rows = row0 + jax.lax.broadcasted_iota(jnp.int32, (KBLK, 128), 0)
    shifts = (L2 - 1) - jax.lax.broadcasted_iota(jnp.int32, (KBLK, 128), 1)
    bv = jnp.where(shifts >= 0,
                   ((rows >> jnp.maximum(shifts, 0)) & 1).astype(jnp.float32),
                   0.0)
    e1 = jnp.maximum(_dot(bv, w1t_ref[...]) + b1_ref[...], 0.0)
    e2 = jnp.maximum(_dot(e1, w2t_ref[...]) + b2_ref[...], 0.0)
    em = _dot(e2, w3t_ref[...]) + b3_ref[...]
    nrm = jnp.sqrt(jnp.sum(em * em, axis=1, keepdims=True))
    en = em / (nrm + 1e-6)
    et_ref[...] = en.T


def _assign_kernel(hin_ref, wpt_ref, bp_ref, et_ref, q_ref, code_ref):
    x = hin_ref[0]
    xt = x.T
    q_ref[0] = xt
    h = _dot(xt, wpt_ref[...]) + bp_ref[...]
    nrm = jnp.sqrt(jnp.sum(h * h, axis=1, keepdims=True))
    hh = h / (nrm + 1e-6)
    hsq = jnp.sum(hh * hh, axis=1, keepdims=True)

    nchunks = K // CHUNK
    big = jnp.float32(jnp.inf)

    def body(c, carry):
        best_v, best_i = carry
        et = et_ref[:, pl.ds(c * CHUNK, CHUNK)]
        esq = jnp.sum(et * et, axis=0, keepdims=True)
        s = _dot(hh, et)
        d = (hsq + esq) - 2.0 * s
        dmin = jnp.min(d, axis=1, keepdims=True)
        idx = jax.lax.broadcasted_iota(jnp.int32, d.shape, 1)
        argc = jnp.min(jnp.where(d == dmin, idx, CHUNK), axis=1,
                       keepdims=True) + c * CHUNK
        upd = dmin < best_v
        return jnp.where(upd, dmin, best_v), jnp.where(upd, argc, best_i)

    init = (jnp.full((T, 1), big, jnp.float32),
            jnp.zeros((T, 1), jnp.int32))
    _, best_i = jax.lax.fori_loop(0, nchunks, body, init)
    code_ref[...] = best_i.reshape(1, 1, T)


def kernel(h_in, W1, b1, W2, b2, W3, b3, Wp, bp, Wpi, bpi):
    del Wpi, bpi
    w1t = jnp.zeros((128, HID), jnp.float32).at[:L2].set(W1.T)
    et = pl.pallas_call(
        _embed_kernel,
        grid=(K // KBLK,),
        in_specs=[
            pl.BlockSpec((128, HID), lambda i: (0, 0)),
            pl.BlockSpec((1, HID), lambda i: (0, 0)),
            pl.BlockSpec((HID, HID), lambda i: (0, 0)),
            pl.BlockSpec((1, HID), lambda i: (0, 0)),
            pl.BlockSpec((HID, VQ), lambda i: (0, 0)),
            pl.BlockSpec((1, VQ), lambda i: (0, 0)),
        ],
        out_specs=pl.BlockSpec((VQ, KBLK), lambda i: (0, i)),
        out_shape=jax.ShapeDtypeStruct((VQ, K), jnp.float32),
        compiler_params=pltpu.CompilerParams(
            dimension_semantics=("arbitrary",)),
    )(w1t, b1.reshape(1, HID), W2.T, b2.reshape(1, HID), W3.T,
      b3.reshape(1, VQ))

    q, code3 = pl.pallas_call(
        _assign_kernel,
        grid=(B,),
        in_specs=[
            pl.BlockSpec((1, EMB, T), lambda b: (b, 0, 0)),
            pl.BlockSpec((EMB, VQ), lambda b: (0, 0)),
            pl.BlockSpec((1, VQ), lambda b: (0, 0)),
            pl.BlockSpec((VQ, K), lambda b: (0, 0)),
        ],
        out_specs=[
            pl.BlockSpec((1, T, EMB), lambda b: (b, 0, 0)),
            pl.BlockSpec((1, 1, T), lambda b: (b, 0, 0)),
        ],
        out_shape=[
            jax.ShapeDtypeStruct((B, T, EMB), jnp.float32),
            jax.ShapeDtypeStruct((B, 1, T), jnp.int32),
        ],
        compiler_params=pltpu.CompilerParams(
            dimension_semantics=("arbitrary",)),
    )(h_in, Wp.T, bp.reshape(1, VQ), et)

    vq_code = code3.reshape(B, T)
    vq_loss = jnp.zeros((), jnp.float32)
    return (q, vq_code, vq_loss)


# speedup vs baseline: 2.8010x; 1.0702x over previous
import functools

import jax
import jax.numpy as jnp
from jax.experimental import pallas as pl
from jax.experimental.pallas import tpu as pltpu

L2 = 13
K = 2 ** L2
VQ = 64
HID = 1024
EMB = 768
B = 16
T = 576

KBLK = 1024
CHUNK = 1024

_dot = functools.partial(jnp.dot, preferred_element_type=jnp.float32,
                         precision=jax.lax.Precision.DEFAULT)


def _embed_kernel(w1t_ref, b1_ref, w2t_ref, b2_ref, w3t_ref, b3_ref,
                  et_ref):
    i = pl.program_id(0)
    row0 = i * KBLK
    rows = row0 + jax.lax.broadcasted_iota(jnp.int32, (KBLK, 128), 0)
    shifts = (L2 - 1) - jax.lax.broadcasted_iota(jnp.int32, (KBLK, 128), 1)
    bv = jnp.where(shifts >= 0,
                   ((rows >> jnp.maximum(shifts, 0)) & 1).astype(jnp.float32),
                   0.0)
    e1 = jnp.maximum(_dot(bv, w1t_ref[...]) + b1_ref[...], 0.0)
    e2 = jnp.maximum(_dot(e1, w2t_ref[...]) + b2_ref[...], 0.0)
    em = _dot(e2, w3t_ref[...]) + b3_ref[...]
    nrm = jnp.sqrt(jnp.sum(em * em, axis=1, keepdims=True))
    en = em / (nrm + 1e-6)
    et_ref[...] = (-2.0) * en.T


def _assign_kernel(hin_ref, wpt_ref, bp_ref, et_ref, q_ref, code_ref):
    x = hin_ref[0]
    xt = x.T
    q_ref[0] = xt
    h = _dot(xt, wpt_ref[...]) + bp_ref[...]
    nrm = jnp.sqrt(jnp.sum(h * h, axis=1, keepdims=True))
    hh = h / (nrm + 1e-6)
    hsq = jnp.sum(hh * hh, axis=1, keepdims=True)

    et = et_ref[...]
    esq = 0.25 * jnp.sum(et * et, axis=0, keepdims=True)
    s2 = _dot(hh, et)
    d = (hsq + esq) + s2
    dmin = jnp.min(d, axis=1, keepdims=True)
    idx = jax.lax.broadcasted_iota(jnp.int32, d.shape, 1)
    best_i = jnp.min(jnp.where(d == dmin, idx, K), axis=1)
    code_ref[...] = best_i.reshape(1, 1, T)


def kernel(h_in, W1, b1, W2, b2, W3, b3, Wp, bp, Wpi, bpi):
    del Wpi, bpi
    w1t = jnp.zeros((128, HID), jnp.float32).at[:L2].set(W1.T)
    et = pl.pallas_call(
        _embed_kernel,
        grid=(K // KBLK,),
        in_specs=[
            pl.BlockSpec((128, HID), lambda i: (0, 0)),
            pl.BlockSpec((1, HID), lambda i: (0, 0)),
            pl.BlockSpec((HID, HID), lambda i: (0, 0)),
            pl.BlockSpec((1, HID), lambda i: (0, 0)),
            pl.BlockSpec((HID, VQ), lambda i: (0, 0)),
            pl.BlockSpec((1, VQ), lambda i: (0, 0)),
        ],
        out_specs=pl.BlockSpec((VQ, KBLK), lambda i: (0, i)),
        out_shape=jax.ShapeDtypeStruct((VQ, K), jnp.float32),
        compiler_params=pltpu.CompilerParams(
            dimension_semantics=("arbitrary",)),
    )(w1t, b1.reshape(1, HID), W2.T, b2.reshape(1, HID), W3.T,
      b3.reshape(1, VQ))

    q, code3 = pl.pallas_call(
        _assign_kernel,
        grid=(B,),
        in_specs=[
            pl.BlockSpec((1, EMB, T), lambda b: (b, 0, 0)),
            pl.BlockSpec((EMB, VQ), lambda b: (0, 0)),
            pl.BlockSpec((1, VQ), lambda b: (0, 0)),
            pl.BlockSpec((VQ, K), lambda b: (0, 0)),
        ],
        out_specs=[
            pl.BlockSpec((1, T, EMB), lambda b: (b, 0, 0)),
            pl.BlockSpec((1, 1, T), lambda b: (b, 0, 0)),
        ],
        out_shape=[
            jax.ShapeDtypeStruct((B, T, EMB), jnp.float32),
            jax.ShapeDtypeStruct((B, 1, T), jnp.int32),
        ],
        compiler_params=pltpu.CompilerParams(
            dimension_semantics=("arbitrary",)),
    )(h_in, Wp.T, bp.reshape(1, VQ), et)

    vq_code = code3.reshape(B, T)
    vq_loss = jnp.zeros((), jnp.float32)
    return (q, vq_code, vq_loss)


# speedup vs baseline: 4.0367x; 1.4412x over previous
import functools

import jax
import jax.numpy as jnp
from jax.experimental import pallas as pl
from jax.experimental.pallas import tpu as pltpu

L2 = 13
K = 2 ** L2
VQ = 64
HID = 1024
EMB = 768
B = 16
T = 576

KBLK = 1024

_dot = functools.partial(jnp.dot, preferred_element_type=jnp.float32,
                         precision=jax.lax.Precision.DEFAULT)


def _dot_t(a, b):
    return jax.lax.dot_general(a, b, (((1,), (1,)), ((), ())),
                               preferred_element_type=jnp.float32,
                               precision=jax.lax.Precision.DEFAULT)


def _fused_kernel(w1t_ref, b1_ref, w2_ref, b2_ref, w3_ref, b3_ref,
                  hin_ref, wp_ref, bp_ref, q_ref, code_ref, et_ref):
    @pl.when(pl.program_id(0) == 0)
    def _embed():
        def chunk(c, carry):
            row0 = c * KBLK
            rows = row0 + jax.lax.broadcasted_iota(jnp.int32, (KBLK, 128), 0)
            shifts = (L2 - 1) - jax.lax.broadcasted_iota(jnp.int32, (KBLK, 128), 1)
            bv = jnp.where(shifts >= 0,
                           ((rows >> jnp.maximum(shifts, 0)) & 1).astype(jnp.float32),
                           0.0)
            e1 = jnp.maximum(_dot(bv, w1t_ref[...]) + b1_ref[...], 0.0)
            e2 = jnp.maximum(_dot_t(e1, w2_ref[...]) + b2_ref[...], 0.0)
            em = _dot_t(e2, w3_ref[...]) + b3_ref[...]
            nrm = jnp.sqrt(jnp.sum(em * em, axis=1, keepdims=True))
            en = em / (nrm + 1e-6)
            et_ref[:, pl.ds(c * KBLK, KBLK)] = (-2.0) * en.T
            return carry
        jax.lax.fori_loop(0, K // KBLK, chunk, 0)

    x = hin_ref[0]
    xt = x.T
    q_ref[0] = xt
    h = _dot_t(xt, wp_ref[...]) + bp_ref[...]
    nrm = jnp.sqrt(jnp.sum(h * h, axis=1, keepdims=True))
    hh = h / (nrm + 1e-6)
    hsq = jnp.sum(hh * hh, axis=1, keepdims=True)
    et = et_ref[...]
    esq = 0.25 * jnp.sum(et * et, axis=0, keepdims=True)
    s2 = _dot(hh, et)
    d = (hsq + esq) + s2
    best_i = jnp.argmin(d, axis=1).astype(jnp.int32)
    code_ref[...] = best_i.reshape(1, 1, T)


def kernel(h_in, W1, b1, W2, b2, W3, b3, Wp, bp, Wpi, bpi):
    del Wpi, bpi
    w1t = jnp.zeros((128, HID), jnp.float32).at[:L2].set(W1.T)
    q, code3 = pl.pallas_call(
        _fused_kernel,
        grid=(B,),
        in_specs=[
            pl.BlockSpec((128, HID), lambda b: (0, 0)),
            pl.BlockSpec((1, HID), lambda b: (0, 0)),
            pl.BlockSpec((HID, HID), lambda b: (0, 0)),
            pl.BlockSpec((1, HID), lambda b: (0, 0)),
            pl.BlockSpec((VQ, HID), lambda b: (0, 0)),
            pl.BlockSpec((1, VQ), lambda b: (0, 0)),
            pl.BlockSpec((1, EMB, T), lambda b: (b, 0, 0)),
            pl.BlockSpec((VQ, EMB), lambda b: (0, 0)),
            pl.BlockSpec((1, VQ), lambda b: (0, 0)),
        ],
        out_specs=[
            pl.BlockSpec((1, T, EMB), lambda b: (b, 0, 0)),
            pl.BlockSpec((1, 1, T), lambda b: (b, 0, 0)),
        ],
        out_shape=[
            jax.ShapeDtypeStruct((B, T, EMB), jnp.float32),
            jax.ShapeDtypeStruct((B, 1, T), jnp.int32),
        ],
        scratch_shapes=[pltpu.VMEM((VQ, K), jnp.float32)],
        compiler_params=pltpu.CompilerParams(
            dimension_semantics=("arbitrary",)),
    )(w1t, b1.reshape(1, HID), W2, b2.reshape(1, HID), W3,
      b3.reshape(1, VQ), h_in, Wp, bp.reshape(1, VQ))

    vq_code = code3.reshape(B, T)
    vq_loss = jnp.zeros((), jnp.float32)
    return (q, vq_code, vq_loss)


# speedup vs baseline: 4.0952x; 1.0145x over previous
import functools

import jax
import jax.numpy as jnp
from jax.experimental import pallas as pl
from jax.experimental.pallas import tpu as pltpu

L2 = 13
K = 2 ** L2
VQ = 64
HID = 1024
EMB = 768
B = 16
T = 576

KBLK = 1024

_dot = functools.partial(jnp.dot, preferred_element_type=jnp.float32,
                         precision=jax.lax.Precision.DEFAULT)


def _dot_t(a, b):
    return jax.lax.dot_general(a, b, (((1,), (1,)), ((), ())),
                               preferred_element_type=jnp.float32,
                               precision=jax.lax.Precision.DEFAULT)


def _fused_kernel(w1t_ref, b1_ref, w2_ref, b2_ref, w3_ref, b3_ref,
                  hin_ref, wp_ref, bp_ref, q_ref, code_ref, et_ref, esq_ref):
    @pl.when(pl.program_id(0) == 0)
    def _embed():
        def chunk(c, carry):
            row0 = c * KBLK
            rows = row0 + jax.lax.broadcasted_iota(jnp.int32, (KBLK, 128), 0)
            shifts = (L2 - 1) - jax.lax.broadcasted_iota(jnp.int32, (KBLK, 128), 1)
            bv = jnp.where(shifts >= 0,
                           ((rows >> jnp.maximum(shifts, 0)) & 1).astype(jnp.float32),
                           0.0)
            e1 = jnp.maximum(_dot(bv, w1t_ref[...]) + b1_ref[...], 0.0)
            e2 = jnp.maximum(_dot_t(e1, w2_ref[...]) + b2_ref[...], 0.0)
            em = _dot_t(e2, w3_ref[...]) + b3_ref[...]
            nrm = jnp.sqrt(jnp.sum(em * em, axis=1, keepdims=True))
            en = em / (nrm + 1e-6)
            et_ref[:, pl.ds(c * KBLK, KBLK)] = (-2.0) * en.T
            return carry
        jax.lax.fori_loop(0, K // KBLK, chunk, 0)
        ets = et_ref[...]
        esq_ref[...] = 0.25 * jnp.sum(ets * ets, axis=0, keepdims=True)

    x = hin_ref[0]
    xt = x.T
    q_ref[0] = xt
    h = _dot_t(xt, wp_ref[...]) + bp_ref[...]
    nrm = jnp.sqrt(jnp.sum(h * h, axis=1, keepdims=True))
    hh = h / (nrm + 1e-6)
    hsq = jnp.sum(hh * hh, axis=1, keepdims=True)
    et = et_ref[...]
    esq = esq_ref[...]
    s2 = _dot(hh, et)
    d = (hsq + esq) + s2
    best_i = jnp.argmin(d, axis=1).astype(jnp.int32)
    code_ref[...] = best_i.reshape(1, 1, T)


def kernel(h_in, W1, b1, W2, b2, W3, b3, Wp, bp, Wpi, bpi):
    del Wpi, bpi
    w1t = jnp.zeros((128, HID), jnp.float32).at[:L2].set(W1.T)
    q, code3 = pl.pallas_call(
        _fused_kernel,
        grid=(B,),
        in_specs=[
            pl.BlockSpec((128, HID), lambda b: (0, 0)),
            pl.BlockSpec((1, HID), lambda b: (0, 0)),
            pl.BlockSpec((HID, HID), lambda b: (0, 0)),
            pl.BlockSpec((1, HID), lambda b: (0, 0)),
            pl.BlockSpec((VQ, HID), lambda b: (0, 0)),
            pl.BlockSpec((1, VQ), lambda b: (0, 0)),
            pl.BlockSpec((1, EMB, T), lambda b: (b, 0, 0)),
            pl.BlockSpec((VQ, EMB), lambda b: (0, 0)),
            pl.BlockSpec((1, VQ), lambda b: (0, 0)),
        ],
        out_specs=[
            pl.BlockSpec((1, T, EMB), lambda b: (b, 0, 0)),
            pl.BlockSpec((1, 1, T), lambda b: (b, 0, 0)),
        ],
        out_shape=[
            jax.ShapeDtypeStruct((B, T, EMB), jnp.float32),
            jax.ShapeDtypeStruct((B, 1, T), jnp.int32),
        ],
        scratch_shapes=[pltpu.VMEM((VQ, K), jnp.float32),
                        pltpu.VMEM((1, K), jnp.float32)],
        compiler_params=pltpu.CompilerParams(
            dimension_semantics=("arbitrary",)),
    )(w1t, b1.reshape(1, HID), W2, b2.reshape(1, HID), W3,
      b3.reshape(1, VQ), h_in, Wp, bp.reshape(1, VQ))

    vq_code = code3.reshape(B, T)
    vq_loss = jnp.zeros((), jnp.float32)
    return (q, vq_code, vq_loss)
